# baseline (speedup 1.0000x reference)
import functools

import jax
import jax.numpy as jnp
from jax import lax
from jax.experimental import pallas as pl
from jax.experimental.pallas import tpu as pltpu
from jax.experimental.pallas import tpu_sc as plsc

N = 10000
NP = 10240
H = 256
NHEADS = 3
E = 160000
NW = 32
RNG = 320
CHUNK = 1000
NCHUNK = E // CHUNK
CPAD = CHUNK + 64
SB = 16
BQ = 10240
RPW = BQ // NW
RCH = 32

_MESH = dict(core_axis_name="c", subcore_axis_name="s", num_cores=2,
             num_subcores=16)


def _tc_body(x_ref, w_ref, alr_ref, h_ref, elr_ref):
    x = x_ref[...]
    h = jnp.dot(x, w_ref[...], preferred_element_type=jnp.float32)
    h_ref[...] = h
    a = alr_ref[...]
    tl = h * a[0:1, :]
    tr = h * a[1:2, :]
    cols = []
    for hd in range(NHEADS):
        cols.append(jnp.sum(tl[:, hd * H:(hd + 1) * H], axis=1)[:, None])
    for hd in range(NHEADS):
        cols.append(jnp.sum(tr[:, hd * H:(hd + 1) * H], axis=1)[:, None])
    cols.append(jnp.zeros((x.shape[0], 2), jnp.float32))
    elr_ref[...] = jnp.concatenate(cols, axis=1)


def _tc_layer1(xp, W, alr):
    blk = 512
    return pl.pallas_call(
        _tc_body,
        grid=(NP // blk,),
        in_specs=[
            pl.BlockSpec((blk, H), lambda i: (i, 0)),
            pl.BlockSpec((H, NHEADS * H), lambda i: (0, 0)),
            pl.BlockSpec((2, NHEADS * H), lambda i: (0, 0)),
        ],
        out_specs=[
            pl.BlockSpec((blk, NHEADS * H), lambda i: (i, 0)),
            pl.BlockSpec((blk, 8), lambda i: (i, 0)),
        ],
        out_shape=[
            jax.ShapeDtypeStruct((NP, NHEADS * H), jnp.float32),
            jax.ShapeDtypeStruct((NP, 8), jnp.float32),
        ],
    )(xp, W, alr)


class _Val:
    def __init__(self, v):
        self.v = v

    def __getitem__(self, idx):
        return self.v


def _tc_body2(g_ref, w_ref, alr_ref, bm_ref, h_ref, elr_ref):
    g = g_ref[...]
    x = (g[0] + g[1] + g[2]) * (1.0 / 3.0) + bm_ref[...]
    _tc_body(_Val(x), w_ref, alr_ref, h_ref, elr_ref)


def _tc_layer2(g, W, alr, bm):
    blk = 512
    return pl.pallas_call(
        _tc_body2,
        grid=(NP // blk,),
        in_specs=[
            pl.BlockSpec((NHEADS, blk, H), lambda i: (0, i, 0)),
            pl.BlockSpec((H, NHEADS * H), lambda i: (0, 0)),
            pl.BlockSpec((2, NHEADS * H), lambda i: (0, 0)),
            pl.BlockSpec((1, H), lambda i: (0, 0)),
        ],
        out_specs=[
            pl.BlockSpec((blk, NHEADS * H), lambda i: (i, 0)),
            pl.BlockSpec((blk, 8), lambda i: (i, 0)),
        ],
        out_shape=[
            jax.ShapeDtypeStruct((NP, NHEADS * H), jnp.float32),
            jax.ShapeDtypeStruct((NP, 8), jnp.float32),
        ],
    )(g, W, alr, bm)


def _iota16():
    return lax.broadcasted_iota(jnp.int32, (16,), 0)


def _sc_gat(h_flat, el0, el1, el2, er0, er1, er2, src, dst):
    @functools.partial(
        pl.kernel,
        out_type=[
            jax.ShapeDtypeStruct((NHEADS, NP, H), jnp.float32),
            jax.ShapeDtypeStruct((NW * NCHUNK * CPAD,), jnp.int32),
        ],
        mesh=plsc.VectorSubcoreMesh(**_MESH),
        scratch_types=dict(
            acc=pltpu.VMEM((RNG, H), jnp.float32),
            den=pltpu.VMEM((RNG, 16), jnp.float32),
            srcv=pltpu.VMEM((CHUNK,), jnp.int32),
            dstv=pltpu.VMEM((CHUNK,), jnp.int32),
            cbuf=pltpu.VMEM((CPAD,), jnp.int32),
            cnts=pltpu.SMEM((NCHUNK,), jnp.int32),
            hidxb=pltpu.VMEM((SB,), jnp.int32),
            srcb=pltpu.VMEM((SB,), jnp.int32),
            dstb=pltpu.VMEM((SB + 16,), jnp.int32),
            elvb=pltpu.VMEM((SB,), jnp.float32),
            ervb=pltpu.VMEM((SB,), jnp.float32),
            wvb=pltpu.VMEM((SB + 16,), jnp.float32),
            hrows=pltpu.VMEM((SB, H), jnp.float32),
            sem1=pltpu.SemaphoreType.DMA,
            sem2=pltpu.SemaphoreType.DMA,
            sem3=pltpu.SemaphoreType.DMA,
        ),
    )
    def gat(h_hbm, el0_h, el1_h, el2_h, er0_h, er1_h, er2_h, src_hbm,
            dst_hbm, g_hbm, pk_hbm, *, acc, den, srcv, dstv, cbuf, cnts,
            hidxb, srcb, dstb, elvb, ervb, wvb, hrows, sem1, sem2, sem3):
        wid = lax.axis_index("c") * 16 + lax.axis_index("s")
        nbase = wid * RNG
        pkbase = wid * (NCHUNK * CPAD)
        i16 = _iota16()
        one16 = jnp.ones((16,), jnp.int32)
        izero16 = jnp.zeros((16,), jnp.int32)
        zero16 = jnp.zeros((16,), jnp.float32)

        def chunk1(ci, _):
            pltpu.sync_copy(src_hbm.at[pl.ds(ci * CHUNK, CHUNK)], srcv)
            pltpu.sync_copy(dst_hbm.at[pl.ds(ci * CHUNK, CHUNK)], dstv)

            def grp(g2, nm):
                sl = pl.ds(g2 * 16, 16)
                dv = dstv[sl]
                off = dv - nbase
                m = (off >= 0) & (off < RNG)
                c01 = jnp.where(m, one16, izero16)
                val = jnp.where(m, srcv[sl] * 16384 + dv, izero16)

                for j in range(16):
                    cbuf[pl.ds(nm, 16)] = jnp.full((16,), val[j],
                                                   jnp.int32)
                    nm = nm + c01[j]
                return nm

            nm = lax.fori_loop(0, CHUNK // 16, grp, 0)
            cnts[ci] = nm
            pltpu.sync_copy(cbuf, pk_hbm.at[pl.ds(pkbase + ci * CPAD, CPAD)])
            return 0

        lax.fori_loop(0, NCHUNK, chunk1, 0)

        for head in range(NHEADS):
            elp = (el0_h, el1_h, el2_h)[head]
            erp = (er0_h, er1_h, er2_h)[head]

            def zbody(n, _):
                for c in range(16):
                    acc[n, pl.ds(c * 16, 16)] = zero16
                den[n] = zero16
                return 0

            lax.fori_loop(0, RNG, zbody, 0)

            def chunk2(ci, _):
                pltpu.sync_copy(pk_hbm.at[pl.ds(pkbase + ci * CPAD, CPAD)],
                                cbuf)
                nm = cnts[ci]
                nb = (nm + (SB - 1)) // SB

                def batch(bi, _):
                    done = bi * SB
                    cnt = jnp.minimum(SB, nm - done)
                    for g2 in range(SB // 16):
                        sl = pl.ds(g2 * 16, 16)
                        v = cbuf[pl.ds(done + g2 * 16, 16)]
                        sv = lax.shift_right_logical(v, 14)
                        dv = v & 16383
                        hidxb[sl] = sv * 3 + head
                        srcb[sl] = sv
                        dstb[sl] = dv
                    cp1 = pltpu.async_copy(h_hbm.at[hidxb], hrows, sem1)
                    cp2 = pltpu.async_copy(elp.at[srcb], elvb, sem2)
                    cp3 = pltpu.async_copy(erp.at[dstb.at[pl.ds(0, SB)]],
                                           ervb, sem3)
                    cp1.wait()
                    cp2.wait()
                    cp3.wait()
                    for g2 in range(SB // 16):
                        sl = pl.ds(g2 * 16, 16)
                        z = elvb[sl] + ervb[sl]
                        wvb[sl] = jnp.exp(jnp.maximum(z, 0.2 * z))

                    def sedge(j, _):
                        dj = dstb[pl.ds(j, 16)][0] - nbase
                        w = jnp.full((16,), wvb[pl.ds(j, 16)][0],
                                     jnp.float32)
                        for c in range(16):
                            sl2 = pl.ds(c * 16, 16)
                            acc[dj, sl2] = acc[dj, sl2] + w * hrows[j, sl2]
                        den[dj] = den[dj] + w
                        return 0

                    lax.fori_loop(0, cnt, sedge, 0)
                    return 0

                lax.fori_loop(0, nb, batch, 0)
                return 0

            lax.fori_loop(0, NCHUNK, chunk2, 0)

            def nbody(n, _):
                inv = 1.0 / jnp.maximum(den[n], 1e-9)
                for c in range(16):
                    acc[n, pl.ds(c * 16, 16)] = acc[n, pl.ds(c * 16, 16)] * inv
                return 0

            lax.fori_loop(0, RNG, nbody, 0)
            pltpu.sync_copy(acc, g_hbm.at[head, pl.ds(nbase, RNG)])

    return gat(h_flat, el0, el1, el2, er0, er1, er2, src, dst)[0]


def _sc_final(g2_flat, nid, sid, rid, ent, rel, globf, bm):
    @functools.partial(
        pl.kernel,
        out_type=[
            jax.ShapeDtypeStruct((BQ, 4 * H), jnp.float32),
            jax.ShapeDtypeStruct((BQ, 3 * H), jnp.float32),
        ],
        mesh=plsc.VectorSubcoreMesh(**_MESH),
        scratch_types=dict(
            idxv=pltpu.VMEM((RCH,), jnp.int32),
            idx2=pltpu.VMEM((RCH,), jnp.int32),
            idx3=pltpu.VMEM((RCH,), jnp.int32),
            bufa=pltpu.VMEM((RCH, H), jnp.float32),
            bufb=pltpu.VMEM((RCH, H), jnp.float32),
            bufc=pltpu.VMEM((RCH, H), jnp.float32),
            entb=pltpu.VMEM((RCH, H), jnp.float32),
            relb=pltpu.VMEM((RCH, H), jnp.float32),
            glob=pltpu.VMEM((RCH, H), jnp.float32),
            st1=pltpu.VMEM((RCH, 4 * H), jnp.float32),
            st2=pltpu.VMEM((RCH, 3 * H), jnp.float32),
            bmv=pltpu.VMEM((1, H), jnp.float32),
            sem1=pltpu.SemaphoreType.DMA,
            sem2=pltpu.SemaphoreType.DMA,
            sem3=pltpu.SemaphoreType.DMA,
        ),
    )
    def fin(g_hbm, nid_hbm, sid_hbm, rid_hbm, ent_hbm, rel_hbm, glob_hbm,
            bm_hbm, out1_hbm, out2_hbm, *, idxv, idx2, idx3, bufa, bufb,
            bufc, entb, relb, glob, st1, st2, bmv, sem1, sem2, sem3):
        wid = lax.axis_index("c") * 16 + lax.axis_index("s")
        pltpu.sync_copy(bm_hbm, bmv)

        def chunk(ch, _):
            rowbase = wid * RPW + ch * RCH
            pltpu.sync_copy(nid_hbm.at[pl.ds(rowbase, RCH)], idxv)
            cpa = pltpu.async_copy(g_hbm.at[idxv], bufa, sem1)
            for g2 in range(RCH // 16):
                sl = pl.ds(g2 * 16, 16)
                idx2[sl] = idxv[sl] + NP
                idx3[sl] = idxv[sl] + 2 * NP
            cpb = pltpu.async_copy(g_hbm.at[idx2], bufb, sem2)
            cpc = pltpu.async_copy(g_hbm.at[idx3], bufc, sem3)
            cpa.wait()
            cpb.wait()
            cpc.wait()
            pltpu.sync_copy(sid_hbm.at[pl.ds(rowbase, RCH)], idx2)
            cpa = pltpu.async_copy(ent_hbm.at[idx2], entb, sem1)
            pltpu.sync_copy(rid_hbm.at[pl.ds(rowbase, RCH)], idx3)
            cpb = pltpu.async_copy(rel_hbm.at[idx3], relb, sem2)
            cpc = pltpu.async_copy(glob_hbm.at[pl.ds(rowbase, RCH)], glob,
                                   sem3)

            def xbody(j, _):
                for c in range(16):
                    sl = pl.ds(c * 16, 16)
                    v = ((bufa[j, sl] + bufb[j, sl] + bufc[j, sl])
                         * (1.0 / 3.0) + bmv[0, sl])
                    st1[j, sl] = v
                    st2[j, sl] = v
                return 0

            lax.fori_loop(0, RCH, xbody, 0)
            cpa.wait()
            cpb.wait()
            cpc.wait()

            def ybody(j, _):
                for c in range(16):
                    sl = pl.ds(c * 16, 16)
                    ev = entb[j, sl]
                    st1[j, pl.ds(H + c * 16, 16)] = ev
                    st2[j, pl.ds(H + c * 16, 16)] = ev
                    st1[j, pl.ds(2 * H + c * 16, 16)] = relb[j, sl]
                    gv = glob[j, sl]
                    st1[j, pl.ds(3 * H + c * 16, 16)] = gv
                    st2[j, pl.ds(2 * H + c * 16, 16)] = gv
                return 0

            lax.fori_loop(0, RCH, ybody, 0)
            pltpu.sync_copy(st1, out1_hbm.at[pl.ds(rowbase, RCH)])
            pltpu.sync_copy(st2, out2_hbm.at[pl.ds(rowbase, RCH)])
            return 0

        lax.fori_loop(0, RPW // RCH, chunk, 0)

    return fin(g2_flat, nid, sid, rid, ent, rel, globf, bm)


def kernel(x, edge_index, node_ids_graph, s, r, ent_embeds, rel_embeds,
           global_emb, W1, a_l1, a_r1, b1, W2, a_l2, a_r2, b2):
    src = edge_index[0].astype(jnp.int32)
    dst = edge_index[1].astype(jnp.int32)
    xp = jnp.pad(x, ((0, NP - N), (0, 0)))
    alr1 = jnp.stack([a_l1.reshape(-1), a_r1.reshape(-1)])
    alr2 = jnp.stack([a_l2.reshape(-1), a_r2.reshape(-1)])
    bm1 = jnp.mean(b1, axis=0).reshape(1, H)
    bm2 = jnp.mean(b2, axis=0).reshape(1, H)

    h1, elr1 = _tc_layer1(xp, W1, alr1)
    g1 = _sc_gat(h1.reshape(NHEADS * NP, H),
                 elr1[:, 0], elr1[:, 1], elr1[:, 2],
                 elr1[:, 3], elr1[:, 4], elr1[:, 5], src, dst)
    h2, elr2 = _tc_layer2(g1, W2, alr2, bm1)
    g2 = _sc_gat(h2.reshape(NHEADS * NP, H),
                 elr2[:, 0], elr2[:, 1], elr2[:, 2],
                 elr2[:, 3], elr2[:, 4], elr2[:, 5], src, dst)

    nid = node_ids_graph.astype(jnp.int32)
    sid = jnp.repeat(s.astype(jnp.int32), 10)
    rid = jnp.repeat(r.astype(jnp.int32), 10)
    out1, out2 = _sc_final(g2.reshape(NHEADS * NP, H), nid, sid, rid,
                           ent_embeds, rel_embeds,
                           global_emb.reshape(BQ, H), bm2)
    return (out1.reshape(1024, 10, 4 * H), out2.reshape(1024, 10, 3 * H))


# speedup vs baseline: 1.3066x; 1.3066x over previous
import functools

import jax
import jax.numpy as jnp
from jax import lax
from jax.experimental import pallas as pl
from jax.experimental.pallas import tpu as pltpu
from jax.experimental.pallas import tpu_sc as plsc

N = 10000
NP = 10240
H = 256
NHEADS = 3
E = 160000
NW = 32
RNG = 320
CHUNK = 1000
NCHUNK = E // CHUNK
CPAD = CHUNK + 64
SB = 16
BQ = 10240
RPW = BQ // NW
RCH = 32

_MESH = dict(core_axis_name="c", subcore_axis_name="s", num_cores=2,
             num_subcores=16)


def _tc_body(x_ref, w_ref, alr_ref, h_ref, elr_ref):
    x = x_ref[...]
    h = jnp.dot(x, w_ref[...], preferred_element_type=jnp.float32)
    h_ref[...] = h
    a = alr_ref[...]
    tl = h * a[0:1, :]
    tr = h * a[1:2, :]
    cols = []
    for hd in range(NHEADS):
        cols.append(jnp.sum(tl[:, hd * H:(hd + 1) * H], axis=1)[:, None])
    for hd in range(NHEADS):
        cols.append(jnp.sum(tr[:, hd * H:(hd + 1) * H], axis=1)[:, None])
    cols.append(jnp.zeros((x.shape[0], 2), jnp.float32))
    elr_ref[...] = jnp.concatenate(cols, axis=1)


def _tc_layer1(xp, W, alr):
    blk = 512
    return pl.pallas_call(
        _tc_body,
        grid=(NP // blk,),
        in_specs=[
            pl.BlockSpec((blk, H), lambda i: (i, 0)),
            pl.BlockSpec((H, NHEADS * H), lambda i: (0, 0)),
            pl.BlockSpec((2, NHEADS * H), lambda i: (0, 0)),
        ],
        out_specs=[
            pl.BlockSpec((blk, NHEADS * H), lambda i: (i, 0)),
            pl.BlockSpec((blk, 8), lambda i: (i, 0)),
        ],
        out_shape=[
            jax.ShapeDtypeStruct((NP, NHEADS * H), jnp.float32),
            jax.ShapeDtypeStruct((NP, 8), jnp.float32),
        ],
    )(xp, W, alr)


class _Val:
    def __init__(self, v):
        self.v = v

    def __getitem__(self, idx):
        return self.v


def _tc_body2(g_ref, w_ref, alr_ref, bm_ref, h_ref, elr_ref):
    g = g_ref[...]
    x = (g[0] + g[1] + g[2]) * (1.0 / 3.0) + bm_ref[...]
    _tc_body(_Val(x), w_ref, alr_ref, h_ref, elr_ref)


def _tc_layer2(g, W, alr, bm):
    blk = 512
    return pl.pallas_call(
        _tc_body2,
        grid=(NP // blk,),
        in_specs=[
            pl.BlockSpec((NHEADS, blk, H), lambda i: (0, i, 0)),
            pl.BlockSpec((H, NHEADS * H), lambda i: (0, 0)),
            pl.BlockSpec((2, NHEADS * H), lambda i: (0, 0)),
            pl.BlockSpec((1, H), lambda i: (0, 0)),
        ],
        out_specs=[
            pl.BlockSpec((blk, NHEADS * H), lambda i: (i, 0)),
            pl.BlockSpec((blk, 8), lambda i: (i, 0)),
        ],
        out_shape=[
            jax.ShapeDtypeStruct((NP, NHEADS * H), jnp.float32),
            jax.ShapeDtypeStruct((NP, 8), jnp.float32),
        ],
    )(g, W, alr, bm)


def _iota16():
    return lax.broadcasted_iota(jnp.int32, (16,), 0)


def _sc_gat(h_flat, el0, el1, el2, er0, er1, er2, src, dst):
    @functools.partial(
        pl.kernel,
        out_type=[
            jax.ShapeDtypeStruct((NHEADS, NP, H), jnp.float32),
            jax.ShapeDtypeStruct((NW * NCHUNK * CPAD,), jnp.int32),
        ],
        mesh=plsc.VectorSubcoreMesh(**_MESH),
        scratch_types=dict(
            acc=pltpu.VMEM((RNG, H), jnp.float32),
            den=pltpu.VMEM((RNG, 16), jnp.float32),
            srcv=pltpu.VMEM((CHUNK,), jnp.int32),
            dstv=pltpu.VMEM((CHUNK,), jnp.int32),
            cbuf=pltpu.VMEM((CPAD,), jnp.int32),
            cnts=pltpu.SMEM((NCHUNK,), jnp.int32),
            hidxb=pltpu.VMEM((SB,), jnp.int32),
            srcb=pltpu.VMEM((SB,), jnp.int32),
            dstb=pltpu.VMEM((SB + 16,), jnp.int32),
            elvb=pltpu.VMEM((SB,), jnp.float32),
            ervb=pltpu.VMEM((SB,), jnp.float32),
            wvb=pltpu.VMEM((SB + 16,), jnp.float32),
            hrows=pltpu.VMEM((SB, H), jnp.float32),
            sem1=pltpu.SemaphoreType.DMA,
            sem2=pltpu.SemaphoreType.DMA,
            sem3=pltpu.SemaphoreType.DMA,
        ),
    )
    def gat(h_hbm, el0_h, el1_h, el2_h, er0_h, er1_h, er2_h, src_hbm,
            dst_hbm, g_hbm, pk_hbm, *, acc, den, srcv, dstv, cbuf, cnts,
            hidxb, srcb, dstb, elvb, ervb, wvb, hrows, sem1, sem2, sem3):
        wid = lax.axis_index("c") * 16 + lax.axis_index("s")
        nbase = wid * RNG
        pkbase = wid * (NCHUNK * CPAD)
        i16 = _iota16()
        one16 = jnp.ones((16,), jnp.int32)
        izero16 = jnp.zeros((16,), jnp.int32)
        zero16 = jnp.zeros((16,), jnp.float32)

        def chunk1(ci, _):
            pltpu.sync_copy(src_hbm.at[pl.ds(ci * CHUNK, CHUNK)], srcv)
            pltpu.sync_copy(dst_hbm.at[pl.ds(ci * CHUNK, CHUNK)], dstv)

            def grp(g2, nm):
                sl = pl.ds(g2 * 16, 16)
                dv = dstv[sl]
                off = dv - nbase
                m = (off >= 0) & (off < RNG)
                c01 = jnp.where(m, one16, izero16)
                val = jnp.where(m, srcv[sl] * 16384 + dv, izero16)

                for j in range(16):
                    cj = c01[j]

                    @pl.when(cj > 0)
                    def _():
                        cbuf[pl.ds(nm, 16)] = jnp.full(
                            (16,), val[j], jnp.int32)

                    nm = nm + cj
                return nm

            nm = lax.fori_loop(0, CHUNK // 16, grp, 0)
            cnts[ci] = nm
            pltpu.sync_copy(cbuf, pk_hbm.at[pl.ds(pkbase + ci * CPAD, CPAD)])
            return 0

        lax.fori_loop(0, NCHUNK, chunk1, 0)

        for head in range(NHEADS):
            elp = (el0_h, el1_h, el2_h)[head]
            erp = (er0_h, er1_h, er2_h)[head]

            def zbody(n, _):
                for c in range(16):
                    acc[n, pl.ds(c * 16, 16)] = zero16
                den[n] = zero16
                return 0

            lax.fori_loop(0, RNG, zbody, 0)

            def chunk2(ci, _):
                pltpu.sync_copy(pk_hbm.at[pl.ds(pkbase + ci * CPAD, CPAD)],
                                cbuf)
                nm = cnts[ci]
                nb = (nm + (SB - 1)) // SB

                def batch(bi, _):
                    done = bi * SB
                    cnt = jnp.minimum(SB, nm - done)
                    for g2 in range(SB // 16):
                        sl = pl.ds(g2 * 16, 16)
                        v = cbuf[pl.ds(done + g2 * 16, 16)]
                        sv = lax.shift_right_logical(v, 14)
                        dv = v & 16383
                        hidxb[sl] = sv * 3 + head
                        srcb[sl] = sv
                        dstb[sl] = dv
                    cp1 = pltpu.async_copy(h_hbm.at[hidxb], hrows, sem1)
                    cp2 = pltpu.async_copy(elp.at[srcb], elvb, sem2)
                    cp3 = pltpu.async_copy(erp.at[dstb.at[pl.ds(0, SB)]],
                                           ervb, sem3)
                    cp1.wait()
                    cp2.wait()
                    cp3.wait()
                    for g2 in range(SB // 16):
                        sl = pl.ds(g2 * 16, 16)
                        z = elvb[sl] + ervb[sl]
                        wvb[sl] = jnp.exp(jnp.maximum(z, 0.2 * z))

                    def sedge(j, _):
                        dj = dstb[pl.ds(j, 16)][0] - nbase
                        w = jnp.full((16,), wvb[pl.ds(j, 16)][0],
                                     jnp.float32)
                        for c in range(16):
                            sl2 = pl.ds(c * 16, 16)
                            acc[dj, sl2] = acc[dj, sl2] + w * hrows[j, sl2]
                        den[dj] = den[dj] + w
                        return 0

                    lax.fori_loop(0, cnt, sedge, 0)
                    return 0

                lax.fori_loop(0, nb, batch, 0)
                return 0

            lax.fori_loop(0, NCHUNK, chunk2, 0)

            def nbody(n, _):
                inv = 1.0 / jnp.maximum(den[n], 1e-9)
                for c in range(16):
                    acc[n, pl.ds(c * 16, 16)] = acc[n, pl.ds(c * 16, 16)] * inv
                return 0

            lax.fori_loop(0, RNG, nbody, 0)
            pltpu.sync_copy(acc, g_hbm.at[head, pl.ds(nbase, RNG)])

    return gat(h_flat, el0, el1, el2, er0, er1, er2, src, dst)[0]


def _sc_final(g2_flat, nid, sid, rid, ent, rel, globf, bm):
    @functools.partial(
        pl.kernel,
        out_type=[
            jax.ShapeDtypeStruct((BQ, 4 * H), jnp.float32),
            jax.ShapeDtypeStruct((BQ, 3 * H), jnp.float32),
        ],
        mesh=plsc.VectorSubcoreMesh(**_MESH),
        scratch_types=dict(
            idxv=pltpu.VMEM((RCH,), jnp.int32),
            idx2=pltpu.VMEM((RCH,), jnp.int32),
            idx3=pltpu.VMEM((RCH,), jnp.int32),
            bufa=pltpu.VMEM((RCH, H), jnp.float32),
            bufb=pltpu.VMEM((RCH, H), jnp.float32),
            bufc=pltpu.VMEM((RCH, H), jnp.float32),
            entb=pltpu.VMEM((RCH, H), jnp.float32),
            relb=pltpu.VMEM((RCH, H), jnp.float32),
            glob=pltpu.VMEM((RCH, H), jnp.float32),
            st1=pltpu.VMEM((RCH, 4 * H), jnp.float32),
            st2=pltpu.VMEM((RCH, 3 * H), jnp.float32),
            bmv=pltpu.VMEM((1, H), jnp.float32),
            sem1=pltpu.SemaphoreType.DMA,
            sem2=pltpu.SemaphoreType.DMA,
            sem3=pltpu.SemaphoreType.DMA,
        ),
    )
    def fin(g_hbm, nid_hbm, sid_hbm, rid_hbm, ent_hbm, rel_hbm, glob_hbm,
            bm_hbm, out1_hbm, out2_hbm, *, idxv, idx2, idx3, bufa, bufb,
            bufc, entb, relb, glob, st1, st2, bmv, sem1, sem2, sem3):
        wid = lax.axis_index("c") * 16 + lax.axis_index("s")
        pltpu.sync_copy(bm_hbm, bmv)

        def chunk(ch, _):
            rowbase = wid * RPW + ch * RCH
            pltpu.sync_copy(nid_hbm.at[pl.ds(rowbase, RCH)], idxv)
            cpa = pltpu.async_copy(g_hbm.at[idxv], bufa, sem1)
            for g2 in range(RCH // 16):
                sl = pl.ds(g2 * 16, 16)
                idx2[sl] = idxv[sl] + NP
                idx3[sl] = idxv[sl] + 2 * NP
            cpb = pltpu.async_copy(g_hbm.at[idx2], bufb, sem2)
            cpc = pltpu.async_copy(g_hbm.at[idx3], bufc, sem3)
            cpa.wait()
            cpb.wait()
            cpc.wait()
            pltpu.sync_copy(sid_hbm.at[pl.ds(rowbase, RCH)], idx2)
            cpa = pltpu.async_copy(ent_hbm.at[idx2], entb, sem1)
            pltpu.sync_copy(rid_hbm.at[pl.ds(rowbase, RCH)], idx3)
            cpb = pltpu.async_copy(rel_hbm.at[idx3], relb, sem2)
            cpc = pltpu.async_copy(glob_hbm.at[pl.ds(rowbase, RCH)], glob,
                                   sem3)

            def xbody(j, _):
                for c in range(16):
                    sl = pl.ds(c * 16, 16)
                    v = ((bufa[j, sl] + bufb[j, sl] + bufc[j, sl])
                         * (1.0 / 3.0) + bmv[0, sl])
                    st1[j, sl] = v
                    st2[j, sl] = v
                return 0

            lax.fori_loop(0, RCH, xbody, 0)
            cpa.wait()
            cpb.wait()
            cpc.wait()

            def ybody(j, _):
                for c in range(16):
                    sl = pl.ds(c * 16, 16)
                    ev = entb[j, sl]
                    st1[j, pl.ds(H + c * 16, 16)] = ev
                    st2[j, pl.ds(H + c * 16, 16)] = ev
                    st1[j, pl.ds(2 * H + c * 16, 16)] = relb[j, sl]
                    gv = glob[j, sl]
                    st1[j, pl.ds(3 * H + c * 16, 16)] = gv
                    st2[j, pl.ds(2 * H + c * 16, 16)] = gv
                return 0

            lax.fori_loop(0, RCH, ybody, 0)
            pltpu.sync_copy(st1, out1_hbm.at[pl.ds(rowbase, RCH)])
            pltpu.sync_copy(st2, out2_hbm.at[pl.ds(rowbase, RCH)])
            return 0

        lax.fori_loop(0, RPW // RCH, chunk, 0)

    return fin(g2_flat, nid, sid, rid, ent, rel, globf, bm)


def kernel(x, edge_index, node_ids_graph, s, r, ent_embeds, rel_embeds,
           global_emb, W1, a_l1, a_r1, b1, W2, a_l2, a_r2, b2):
    src = edge_index[0].astype(jnp.int32)
    dst = edge_index[1].astype(jnp.int32)
    xp = jnp.pad(x, ((0, NP - N), (0, 0)))
    alr1 = jnp.stack([a_l1.reshape(-1), a_r1.reshape(-1)])
    alr2 = jnp.stack([a_l2.reshape(-1), a_r2.reshape(-1)])
    bm1 = jnp.mean(b1, axis=0).reshape(1, H)
    bm2 = jnp.mean(b2, axis=0).reshape(1, H)

    h1, elr1 = _tc_layer1(xp, W1, alr1)
    g1 = _sc_gat(h1.reshape(NHEADS * NP, H),
                 elr1[:, 0], elr1[:, 1], elr1[:, 2],
                 elr1[:, 3], elr1[:, 4], elr1[:, 5], src, dst)
    h2, elr2 = _tc_layer2(g1, W2, alr2, bm1)
    g2 = _sc_gat(h2.reshape(NHEADS * NP, H),
                 elr2[:, 0], elr2[:, 1], elr2[:, 2],
                 elr2[:, 3], elr2[:, 4], elr2[:, 5], src, dst)

    nid = node_ids_graph.astype(jnp.int32)
    sid = jnp.repeat(s.astype(jnp.int32), 10)
    rid = jnp.repeat(r.astype(jnp.int32), 10)
    out1, out2 = _sc_final(g2.reshape(NHEADS * NP, H), nid, sid, rid,
                           ent_embeds, rel_embeds,
                           global_emb.reshape(BQ, H), bm2)
    return (out1.reshape(1024, 10, 4 * H), out2.reshape(1024, 10, 3 * H))


# speedup vs baseline: 1.3857x; 1.0605x over previous
import functools

import jax
import jax.numpy as jnp
from jax import lax
from jax.experimental import pallas as pl
from jax.experimental.pallas import tpu as pltpu
from jax.experimental.pallas import tpu_sc as plsc

N = 10000
NP = 10240
H = 256
NHEADS = 3
E = 160000
NW = 32
RNG = 320
CHUNK = 1000
NCHUNK = E // CHUNK
CPAD = CHUNK + 64
SB = 16
BQ = 10240
RPW = BQ // NW
RCH = 32

_MESH = dict(core_axis_name="c", subcore_axis_name="s", num_cores=2,
             num_subcores=16)


def _tc_body(x_ref, w_ref, alr_ref, h_ref, elr_ref):
    x = x_ref[...]
    h = jnp.dot(x, w_ref[...], preferred_element_type=jnp.float32)
    h_ref[...] = h
    a = alr_ref[...]
    tl = h * a[0:1, :]
    tr = h * a[1:2, :]
    cols = []
    for hd in range(NHEADS):
        cols.append(jnp.sum(tl[:, hd * H:(hd + 1) * H], axis=1)[:, None])
    for hd in range(NHEADS):
        cols.append(jnp.sum(tr[:, hd * H:(hd + 1) * H], axis=1)[:, None])
    cols.append(jnp.zeros((x.shape[0], 2), jnp.float32))
    elr_ref[...] = jnp.concatenate(cols, axis=1)


def _tc_layer1(xp, W, alr):
    blk = 512
    return pl.pallas_call(
        _tc_body,
        grid=(NP // blk,),
        in_specs=[
            pl.BlockSpec((blk, H), lambda i: (i, 0)),
            pl.BlockSpec((H, NHEADS * H), lambda i: (0, 0)),
            pl.BlockSpec((2, NHEADS * H), lambda i: (0, 0)),
        ],
        out_specs=[
            pl.BlockSpec((blk, NHEADS * H), lambda i: (i, 0)),
            pl.BlockSpec((blk, 8), lambda i: (i, 0)),
        ],
        out_shape=[
            jax.ShapeDtypeStruct((NP, NHEADS * H), jnp.float32),
            jax.ShapeDtypeStruct((NP, 8), jnp.float32),
        ],
    )(xp, W, alr)


class _Val:
    def __init__(self, v):
        self.v = v

    def __getitem__(self, idx):
        return self.v


def _tc_body2(g_ref, w_ref, alr_ref, bm_ref, h_ref, elr_ref):
    g = g_ref[...]
    x = (g[0] + g[1] + g[2]) * (1.0 / 3.0) + bm_ref[...]
    _tc_body(_Val(x), w_ref, alr_ref, h_ref, elr_ref)


def _tc_layer2(g, W, alr, bm):
    blk = 512
    return pl.pallas_call(
        _tc_body2,
        grid=(NP // blk,),
        in_specs=[
            pl.BlockSpec((NHEADS, blk, H), lambda i: (0, i, 0)),
            pl.BlockSpec((H, NHEADS * H), lambda i: (0, 0)),
            pl.BlockSpec((2, NHEADS * H), lambda i: (0, 0)),
            pl.BlockSpec((1, H), lambda i: (0, 0)),
        ],
        out_specs=[
            pl.BlockSpec((blk, NHEADS * H), lambda i: (i, 0)),
            pl.BlockSpec((blk, 8), lambda i: (i, 0)),
        ],
        out_shape=[
            jax.ShapeDtypeStruct((NP, NHEADS * H), jnp.float32),
            jax.ShapeDtypeStruct((NP, 8), jnp.float32),
        ],
    )(g, W, alr, bm)


def _iota16():
    return lax.broadcasted_iota(jnp.int32, (16,), 0)


def _sc_gat(h_flat, el0, el1, el2, er0, er1, er2, src, dst, bucket=None):
    build = bucket is None
    out_t = [jax.ShapeDtypeStruct((NHEADS, NP, H), jnp.float32)]
    if build:
        out_t.append(jax.ShapeDtypeStruct((NW * NCHUNK * CPAD,),
                                          jnp.int32))

    @functools.partial(
        pl.kernel,
        out_type=out_t,
        mesh=plsc.VectorSubcoreMesh(**_MESH),
        scratch_types=dict(
            acc=pltpu.VMEM((RNG, H), jnp.float32),
            den=pltpu.VMEM((RNG, 16), jnp.float32),
            srcv=pltpu.VMEM((CHUNK,), jnp.int32),
            dstv=pltpu.VMEM((CHUNK,), jnp.int32),
            cbuf=pltpu.VMEM((CPAD,), jnp.int32),
            hidxb=pltpu.VMEM((SB,), jnp.int32),
            srcb=pltpu.VMEM((SB,), jnp.int32),
            dstb=pltpu.VMEM((SB + 16,), jnp.int32),
            elvb=pltpu.VMEM((SB,), jnp.float32),
            ervb=pltpu.VMEM((SB,), jnp.float32),
            wvb=pltpu.VMEM((SB + 16,), jnp.float32),
            hrows=pltpu.VMEM((SB, H), jnp.float32),
            sem1=pltpu.SemaphoreType.DMA,
            sem2=pltpu.SemaphoreType.DMA,
            sem3=pltpu.SemaphoreType.DMA,
        ),
    )
    def gat(h_hbm, el0_h, el1_h, el2_h, er0_h, er1_h, er2_h, src_hbm,
            dst_hbm, *args, acc, den, srcv, dstv, cbuf,
            hidxb, srcb, dstb, elvb, ervb, wvb, hrows, sem1, sem2, sem3):
        if build:
            g_hbm, pk_hbm = args
        else:
            pk_hbm, g_hbm = args
        wid = lax.axis_index("c") * 16 + lax.axis_index("s")
        nbase = wid * RNG
        pkbase = wid * (NCHUNK * CPAD)
        i16 = _iota16()
        one16 = jnp.ones((16,), jnp.int32)
        izero16 = jnp.zeros((16,), jnp.int32)
        zero16 = jnp.zeros((16,), jnp.float32)

        def chunk1(ci, _):
            del _
            pltpu.sync_copy(src_hbm.at[pl.ds(ci * CHUNK, CHUNK)], srcv)
            pltpu.sync_copy(dst_hbm.at[pl.ds(ci * CHUNK, CHUNK)], dstv)

            def grp(g2, nm):
                sl = pl.ds(g2 * 16, 16)
                dv = dstv[sl]
                off = dv - nbase
                m = (off >= 0) & (off < RNG)
                c01 = jnp.where(m, one16, izero16)
                val = jnp.where(m, srcv[sl] * 16384 + dv, izero16)

                for j in range(16):
                    cj = c01[j]

                    @pl.when(cj > 0)
                    def _():
                        cbuf[pl.ds(nm, 16)] = jnp.full(
                            (16,), val[j], jnp.int32)

                    nm = nm + cj
                return nm

            nm = lax.fori_loop(0, CHUNK // 16, grp, 0)
            cbuf[pl.ds(CPAD - 16, 16)] = jnp.full((16,), nm, jnp.int32)
            pltpu.sync_copy(cbuf, pk_hbm.at[pl.ds(pkbase + ci * CPAD, CPAD)])
            return 0

        if build:
            lax.fori_loop(0, NCHUNK, chunk1, 0)

        for head in range(NHEADS):
            elp = (el0_h, el1_h, el2_h)[head]
            erp = (er0_h, er1_h, er2_h)[head]

            def zbody(n, _):
                for c in range(16):
                    acc[n, pl.ds(c * 16, 16)] = zero16
                den[n] = zero16
                return 0

            lax.fori_loop(0, RNG, zbody, 0)

            def chunk2(ci, _):
                pltpu.sync_copy(pk_hbm.at[pl.ds(pkbase + ci * CPAD, CPAD)],
                                cbuf)
                nm = cbuf[pl.ds(CPAD - 16, 16)][0]
                nb = (nm + (SB - 1)) // SB

                def batch(bi, _):
                    done = bi * SB
                    cnt = jnp.minimum(SB, nm - done)
                    for g2 in range(SB // 16):
                        sl = pl.ds(g2 * 16, 16)
                        v = cbuf[pl.ds(done + g2 * 16, 16)]
                        sv = lax.shift_right_logical(v, 14)
                        dv = v & 16383
                        hidxb[sl] = sv * 3 + head
                        srcb[sl] = sv
                        dstb[sl] = dv
                    cp1 = pltpu.async_copy(h_hbm.at[hidxb], hrows, sem1)
                    cp2 = pltpu.async_copy(elp.at[srcb], elvb, sem2)
                    cp3 = pltpu.async_copy(erp.at[dstb.at[pl.ds(0, SB)]],
                                           ervb, sem3)
                    cp1.wait()
                    cp2.wait()
                    cp3.wait()
                    for g2 in range(SB // 16):
                        sl = pl.ds(g2 * 16, 16)
                        z = elvb[sl] + ervb[sl]
                        wvb[sl] = jnp.exp(jnp.maximum(z, 0.2 * z))

                    def sedge(j, _):
                        dj = dstb[pl.ds(j, 16)][0] - nbase
                        w = jnp.full((16,), wvb[pl.ds(j, 16)][0],
                                     jnp.float32)
                        for c in range(16):
                            sl2 = pl.ds(c * 16, 16)
                            acc[dj, sl2] = acc[dj, sl2] + w * hrows[j, sl2]
                        den[dj] = den[dj] + w
                        return 0

                    lax.fori_loop(0, cnt, sedge, 0)
                    return 0

                lax.fori_loop(0, nb, batch, 0)
                return 0

            lax.fori_loop(0, NCHUNK, chunk2, 0)

            def nbody(n, _):
                inv = 1.0 / jnp.maximum(den[n], 1e-9)
                for c in range(16):
                    acc[n, pl.ds(c * 16, 16)] = acc[n, pl.ds(c * 16, 16)] * inv
                return 0

            lax.fori_loop(0, RNG, nbody, 0)
            pltpu.sync_copy(acc, g_hbm.at[head, pl.ds(nbase, RNG)])

    if build:
        g, pk = gat(h_flat, el0, el1, el2, er0, er1, er2, src, dst)
        return g, pk
    return gat(h_flat, el0, el1, el2, er0, er1, er2, src, dst, bucket)[0], None


def _sc_final(g2_flat, nid, sid, rid, ent, rel, globf, bm):
    @functools.partial(
        pl.kernel,
        out_type=[
            jax.ShapeDtypeStruct((BQ, 4 * H), jnp.float32),
            jax.ShapeDtypeStruct((BQ, 3 * H), jnp.float32),
        ],
        mesh=plsc.VectorSubcoreMesh(**_MESH),
        scratch_types=dict(
            idxv=pltpu.VMEM((RCH,), jnp.int32),
            idx2=pltpu.VMEM((RCH,), jnp.int32),
            idx3=pltpu.VMEM((RCH,), jnp.int32),
            bufa=pltpu.VMEM((RCH, H), jnp.float32),
            bufb=pltpu.VMEM((RCH, H), jnp.float32),
            bufc=pltpu.VMEM((RCH, H), jnp.float32),
            entb=pltpu.VMEM((RCH, H), jnp.float32),
            relb=pltpu.VMEM((RCH, H), jnp.float32),
            glob=pltpu.VMEM((RCH, H), jnp.float32),
            st1=pltpu.VMEM((RCH, 4 * H), jnp.float32),
            st2=pltpu.VMEM((RCH, 3 * H), jnp.float32),
            bmv=pltpu.VMEM((1, H), jnp.float32),
            sem1=pltpu.SemaphoreType.DMA,
            sem2=pltpu.SemaphoreType.DMA,
            sem3=pltpu.SemaphoreType.DMA,
        ),
    )
    def fin(g_hbm, nid_hbm, sid_hbm, rid_hbm, ent_hbm, rel_hbm, glob_hbm,
            bm_hbm, out1_hbm, out2_hbm, *, idxv, idx2, idx3, bufa, bufb,
            bufc, entb, relb, glob, st1, st2, bmv, sem1, sem2, sem3):
        wid = lax.axis_index("c") * 16 + lax.axis_index("s")
        pltpu.sync_copy(bm_hbm, bmv)

        def chunk(ch, _):
            rowbase = wid * RPW + ch * RCH
            pltpu.sync_copy(nid_hbm.at[pl.ds(rowbase, RCH)], idxv)
            cpa = pltpu.async_copy(g_hbm.at[idxv], bufa, sem1)
            for g2 in range(RCH // 16):
                sl = pl.ds(g2 * 16, 16)
                idx2[sl] = idxv[sl] + NP
                idx3[sl] = idxv[sl] + 2 * NP
            cpb = pltpu.async_copy(g_hbm.at[idx2], bufb, sem2)
            cpc = pltpu.async_copy(g_hbm.at[idx3], bufc, sem3)
            cpa.wait()
            cpb.wait()
            cpc.wait()
            pltpu.sync_copy(sid_hbm.at[pl.ds(rowbase, RCH)], idx2)
            cpa = pltpu.async_copy(ent_hbm.at[idx2], entb, sem1)
            pltpu.sync_copy(rid_hbm.at[pl.ds(rowbase, RCH)], idx3)
            cpb = pltpu.async_copy(rel_hbm.at[idx3], relb, sem2)
            cpc = pltpu.async_copy(glob_hbm.at[pl.ds(rowbase, RCH)], glob,
                                   sem3)

            def xbody(j, _):
                for c in range(16):
                    sl = pl.ds(c * 16, 16)
                    v = ((bufa[j, sl] + bufb[j, sl] + bufc[j, sl])
                         * (1.0 / 3.0) + bmv[0, sl])
                    st1[j, sl] = v
                    st2[j, sl] = v
                return 0

            lax.fori_loop(0, RCH, xbody, 0)
            cpa.wait()
            cpb.wait()
            cpc.wait()

            def ybody(j, _):
                for c in range(16):
                    sl = pl.ds(c * 16, 16)
                    ev = entb[j, sl]
                    st1[j, pl.ds(H + c * 16, 16)] = ev
                    st2[j, pl.ds(H + c * 16, 16)] = ev
                    st1[j, pl.ds(2 * H + c * 16, 16)] = relb[j, sl]
                    gv = glob[j, sl]
                    st1[j, pl.ds(3 * H + c * 16, 16)] = gv
                    st2[j, pl.ds(2 * H + c * 16, 16)] = gv
                return 0

            lax.fori_loop(0, RCH, ybody, 0)
            pltpu.sync_copy(st1, out1_hbm.at[pl.ds(rowbase, RCH)])
            pltpu.sync_copy(st2, out2_hbm.at[pl.ds(rowbase, RCH)])
            return 0

        lax.fori_loop(0, RPW // RCH, chunk, 0)

    return fin(g2_flat, nid, sid, rid, ent, rel, globf, bm)


def kernel(x, edge_index, node_ids_graph, s, r, ent_embeds, rel_embeds,
           global_emb, W1, a_l1, a_r1, b1, W2, a_l2, a_r2, b2):
    src = edge_index[0].astype(jnp.int32)
    dst = edge_index[1].astype(jnp.int32)
    xp = jnp.pad(x, ((0, NP - N), (0, 0)))
    alr1 = jnp.stack([a_l1.reshape(-1), a_r1.reshape(-1)])
    alr2 = jnp.stack([a_l2.reshape(-1), a_r2.reshape(-1)])
    bm1 = jnp.mean(b1, axis=0).reshape(1, H)
    bm2 = jnp.mean(b2, axis=0).reshape(1, H)

    h1, elr1 = _tc_layer1(xp, W1, alr1)
    g1, pk = _sc_gat(h1.reshape(NHEADS * NP, H),
                     elr1[:, 0], elr1[:, 1], elr1[:, 2],
                     elr1[:, 3], elr1[:, 4], elr1[:, 5], src, dst)
    h2, elr2 = _tc_layer2(g1, W2, alr2, bm1)
    g2, _ = _sc_gat(h2.reshape(NHEADS * NP, H),
                    elr2[:, 0], elr2[:, 1], elr2[:, 2],
                    elr2[:, 3], elr2[:, 4], elr2[:, 5], src, dst,
                    bucket=pk)

    nid = node_ids_graph.astype(jnp.int32)
    sid = jnp.repeat(s.astype(jnp.int32), 10)
    rid = jnp.repeat(r.astype(jnp.int32), 10)
    out1, out2 = _sc_final(g2.reshape(NHEADS * NP, H), nid, sid, rid,
                           ent_embeds, rel_embeds,
                           global_emb.reshape(BQ, H), bm2)
    return (out1.reshape(1024, 10, 4 * H), out2.reshape(1024, 10, 3 * H))
